# baseline (speedup 1.0000x reference)
import functools

import jax
import jax.numpy as jnp
from jax.experimental import pallas as pl
from jax.experimental.pallas import tpu as pltpu

_E = 8
_K = 2
_TB = 128
_RB = 256


def _router_body(xb_ref, wgp_ref, gates_ref, lb_ref, acc_ref, *, n_tokens):
    i = pl.program_id(0)
    nsteps = n_tokens // _RB
    lg = jax.lax.dot_general(
        xb_ref[...], wgp_ref[...], (((1,), (1,)), ((), ())),
        preferred_element_type=jnp.float32)
    lane = jax.lax.broadcasted_iota(jnp.int32, lg.shape, 1)
    neg = jnp.float32(-1e30)
    lgm = jnp.where(lane < _E, lg, neg)
    m1 = jnp.max(lgm, axis=1, keepdims=True)
    i1 = jnp.min(jnp.where(lgm == m1, lane, 127), axis=1, keepdims=True)
    lg2 = jnp.where(lane == i1, neg, lgm)
    m2 = jnp.max(lg2, axis=1, keepdims=True)
    i2 = jnp.min(jnp.where(lg2 == m2, lane, 127), axis=1, keepdims=True)
    t = jnp.exp(m2 - m1)
    g1 = 1.0 / (1.0 + t)
    g2 = t / (1.0 + t)
    gates = jnp.where(lane == i1, g1, 0.0) + jnp.where(lane == i2, g2, 0.0)
    gates_ref[...] = gates

    @pl.when(i == 0)
    def _init():
        acc_ref[...] = jnp.zeros_like(acc_ref)

    acc_ref[0:1, :] += jnp.sum(gates, axis=0, keepdims=True)
    acc_ref[1:2, :] += jnp.sum((gates > 0).astype(jnp.float32), axis=0,
                               keepdims=True)

    @pl.when(i == nsteps - 1)
    def _fin():
        scale = jnp.float32(_E / (float(n_tokens) * float(n_tokens)))
        lb = jnp.sum(acc_ref[0:1, :] * acc_ref[1:2, :], axis=1, keepdims=True)
        lb_ref[...] = lb * scale


def _expert_body(be_ref, xs_ref, w1_ref, b1_ref, w2_ref, b2_ref, w3_ref,
                 b3_ref, sw_ref, ys_ref):
    del be_ref
    xb = xs_ref[...]
    h1 = jax.lax.dot_general(xb, w1_ref[0], (((1,), (1,)), ((), ())),
                             preferred_element_type=jnp.float32)
    h1 = jnp.maximum(h1 + b1_ref[0], 0.0)
    h2 = jax.lax.dot_general(h1, w2_ref[0], (((1,), (1,)), ((), ())),
                             preferred_element_type=jnp.float32)
    h2 = jnp.maximum(h2 + b2_ref[0], 0.0)
    ob = jax.lax.dot_general(h2, w3_ref[0], (((1,), (1,)), ((), ())),
                             preferred_element_type=jnp.float32)
    ys_ref[...] = (ob + b3_ref[0]) * sw_ref[...]


def kernel(x, Wg, W1, b1, W2, b2, W3, b3):
    Bv, Sv, D = x.shape
    N = Bv * Sv
    E, H = W1.shape[0], W1.shape[1]
    O = W3.shape[1]
    xf = x.reshape(N, D)

    wgp = jnp.zeros((128, D), jnp.float32).at[:E].set(Wg)
    gates128, lb = pl.pallas_call(
        functools.partial(_router_body, n_tokens=N),
        grid=(N // _RB,),
        in_specs=[
            pl.BlockSpec((_RB, D), lambda i: (i, 0)),
            pl.BlockSpec((128, D), lambda i: (0, 0)),
        ],
        out_specs=[
            pl.BlockSpec((_RB, 128), lambda i: (i, 0)),
            pl.BlockSpec((1, 1), lambda i: (0, 0)),
        ],
        out_shape=[
            jax.ShapeDtypeStruct((N, 128), jnp.float32),
            jax.ShapeDtypeStruct((1, 1), jnp.float32),
        ],
        scratch_shapes=[pltpu.VMEM((2, 128), jnp.float32)],
    )(xf, wgp)
    gates = gates128[:, :E]
    lb_loss = lb.reshape(())

    GMAX = (N * _K) // _TB + E
    NS = GMAX * _TB
    gt = gates.T
    member = gt > 0.0
    memi = member.astype(jnp.int32)
    rank = jnp.cumsum(memi, axis=1) - 1
    cnt = jnp.sum(memi, axis=1)
    nblk = (cnt + _TB - 1) // _TB
    blk_end = jnp.cumsum(nblk)
    blk_start = blk_end - nblk
    pos = blk_start[:, None] * _TB + rank
    posm = jnp.where(member, pos, NS)
    tok = jnp.broadcast_to(jnp.arange(N, dtype=jnp.int32)[None, :], (E, N))
    slot_token = (jnp.zeros((NS + 1,), jnp.int32)
                  .at[posm.reshape(-1)].set(tok.reshape(-1)))[:NS]
    slot_weight = (jnp.zeros((NS + 1,), jnp.float32)
                   .at[posm.reshape(-1)].set(gt.reshape(-1)))[:NS]
    bid = jnp.arange(GMAX, dtype=jnp.int32)
    be = jnp.sum((bid[:, None] >= blk_end[None, :]).astype(jnp.int32), axis=1)
    be = jnp.minimum(be, E - 1)

    xs = jnp.take(xf, slot_token, axis=0)
    sw2d = slot_weight.reshape(NS, 1)
    grid_spec = pltpu.PrefetchScalarGridSpec(
        num_scalar_prefetch=1,
        grid=(GMAX,),
        in_specs=[
            pl.BlockSpec((_TB, D), lambda g, be_r: (g, 0)),
            pl.BlockSpec((1, H, D), lambda g, be_r: (be_r[g], 0, 0)),
            pl.BlockSpec((1, 1, H), lambda g, be_r: (be_r[g], 0, 0)),
            pl.BlockSpec((1, H, H), lambda g, be_r: (be_r[g], 0, 0)),
            pl.BlockSpec((1, 1, H), lambda g, be_r: (be_r[g], 0, 0)),
            pl.BlockSpec((1, O, H), lambda g, be_r: (be_r[g], 0, 0)),
            pl.BlockSpec((1, 1, O), lambda g, be_r: (be_r[g], 0, 0)),
            pl.BlockSpec((_TB, 1), lambda g, be_r: (g, 0)),
        ],
        out_specs=pl.BlockSpec((_TB, O), lambda g, be_r: (g, 0)),
    )
    ys = pl.pallas_call(
        _expert_body,
        grid_spec=grid_spec,
        out_shape=jax.ShapeDtypeStruct((NS, O), jnp.float32),
        compiler_params=pltpu.CompilerParams(
            dimension_semantics=("arbitrary",)),
    )(be, xs, W1, b1.reshape(E, 1, H), W2, b2.reshape(E, 1, H), W3,
      b3.reshape(E, 1, O), sw2d)

    qpos = posm.T
    q1 = jnp.min(qpos, axis=1)
    q2 = jnp.min(jnp.where(qpos == q1[:, None], NS, qpos), axis=1)
    q1 = jnp.minimum(q1, NS - 1)
    q2 = jnp.where(q2 >= NS, NS - 1, q2)
    out = jnp.take(ys, q1, axis=0) + jnp.take(ys, q2, axis=0)
    return out.reshape(Bv, Sv, O), gates, lb_loss


# speedup vs baseline: 1.4036x; 1.4036x over previous
import functools

import jax
import jax.numpy as jnp
from jax import lax
from jax.experimental import pallas as pl
from jax.experimental.pallas import tpu as pltpu
from jax.experimental.pallas import tpu_sc as plsc

_E = 8
_K = 2
_TB = 128
_RB = 256
_L = 16


def _router_body(xb_ref, wgp_ref, gates_ref, lb_ref, i1_ref, i2_ref,
                 g1_ref, g2_ref, acc_ref, *, n_tokens):
    i = pl.program_id(0)
    nsteps = n_tokens // _RB
    lg = jax.lax.dot_general(
        xb_ref[...], wgp_ref[...], (((1,), (1,)), ((), ())),
        preferred_element_type=jnp.float32)
    lane = jax.lax.broadcasted_iota(jnp.int32, lg.shape, 1)
    neg = jnp.float32(-1e30)
    lgm = jnp.where(lane < _E, lg, neg)
    m1 = jnp.max(lgm, axis=1, keepdims=True)
    i1 = jnp.min(jnp.where(lgm == m1, lane, 127), axis=1, keepdims=True)
    lg2 = jnp.where(lane == i1, neg, lgm)
    m2 = jnp.max(lg2, axis=1, keepdims=True)
    i2 = jnp.min(jnp.where(lg2 == m2, lane, 127), axis=1, keepdims=True)
    t = jnp.exp(m2 - m1)
    g1 = 1.0 / (1.0 + t)
    g2 = t / (1.0 + t)
    gates = jnp.where(lane == i1, g1, 0.0) + jnp.where(lane == i2, g2, 0.0)
    gates_ref[...] = gates
    i1_ref[...] = i1
    i2_ref[...] = i2
    g1_ref[...] = g1
    g2_ref[...] = g2

    @pl.when(i == 0)
    def _init():
        acc_ref[...] = jnp.zeros_like(acc_ref)

    acc_ref[0:1, :] += jnp.sum(gates, axis=0, keepdims=True)
    acc_ref[1:2, :] += jnp.sum((gates > 0).astype(jnp.float32), axis=0,
                               keepdims=True)

    @pl.when(i == nsteps - 1)
    def _fin():
        scale = jnp.float32(_E / (float(n_tokens) * float(n_tokens)))
        lb = jnp.sum(acc_ref[0:1, :] * acc_ref[1:2, :], axis=1, keepdims=True)
        lb_ref[...] = lb * scale


def _dispatch_body(i1_hbm, i2_hbm, g1_hbm, g2_hbm,
                   st_hbm, sw_hbm, q1_hbm, q2_hbm, be_hbm,
                   v_i1, v_i2, v_g1, v_g2, v_st, v_sw, v_q1, v_q2,
                   v_cnt, v_cnts, v_qacc, v_be, sh_cnt, sh_q,
                   *, n_tokens, gmax_pad):
    n_vregs = n_tokens // _L
    cid = lax.axis_index("c")
    sid = lax.axis_index("s")
    core0 = cid == 0
    is_expert = core0 & (sid < _E)
    iota = lax.iota(jnp.int32, _L)

    @pl.when(is_expert)
    def _count():
        pltpu.sync_copy(i1_hbm, v_i1)
        pltpu.sync_copy(i2_hbm, v_i2)
        pltpu.sync_copy(g1_hbm, v_g1)
        pltpu.sync_copy(g2_hbm, v_g2)

        def body(k, cnt):
            a = v_i1[pl.ds(k * _L, _L)]
            b = v_i2[pl.ds(k * _L, _L)]
            c1 = jnp.where(a == sid, 1, 0)
            c2 = jnp.where(b == sid, 1, 0)
            return cnt + jnp.sum(c1 + c2)

        cnt = lax.fori_loop(0, n_vregs, body, jnp.int32(0))
        v_cnt[...] = jnp.broadcast_to(cnt, (_L,))
        pltpu.sync_copy(v_cnt, sh_cnt.at[sid])

    plsc.subcore_barrier()

    @pl.when(core0 & (sid < 11))
    def _layout():
        pltpu.sync_copy(sh_cnt, v_cnts)

    plsc.subcore_barrier()

    @pl.when(is_expert)
    def _place():
        base = jnp.int32(0)
        for e in range(_E):
            ce = jnp.max(v_cnts[e])
            nb = (ce + (_TB - 1)) // _TB
            base = base + jnp.where(e < sid, nb * _TB, 0)

        def scan(k, off, src_ref, wsrc_ref, q_ref):
            a = src_ref[pl.ds(k * _L, _L)]
            w = wsrc_ref[pl.ds(k * _L, _L)]
            m = a == sid
            pref = jnp.cumsum(jnp.where(m, 1, 0))
            loc = off + pref - 1
            tokv = iota + k * _L
            plsc.store_scatter(v_st, [loc], tokv, mask=m)
            plsc.store_scatter(v_sw, [loc], w, mask=m)
            q_ref[pl.ds(k * _L, _L)] = jnp.where(m, base + loc, 0)
            return off + jnp.max(pref)

        off = lax.fori_loop(
            0, n_vregs, lambda k, o: scan(k, o, v_i1, v_g1, v_q1),
            jnp.int32(0))
        off = lax.fori_loop(
            0, n_vregs, lambda k, o: scan(k, o, v_i2, v_g2, v_q2), off)

        nblk = (off + (_TB - 1)) // _TB
        for j in range(n_tokens // _TB):
            @pl.when(j < nblk)
            def _wr():
                dst = pl.multiple_of(base + j * _TB, _TB)
                pltpu.sync_copy(v_st.at[pl.ds(j * _TB, _TB)],
                                st_hbm.at[pl.ds(dst, _TB)])
                pltpu.sync_copy(v_sw.at[pl.ds(j * _TB, _TB)],
                                sw_hbm.at[pl.ds(dst, _TB)])

        pltpu.sync_copy(v_q1, sh_q.at[0, sid])
        pltpu.sync_copy(v_q2, sh_q.at[1, sid])

    @pl.when(core0 & (sid == 10))
    def _blockmap():
        for j in range(gmax_pad // _L):
            bid = iota + j * _L
            acc = jnp.int32(0)
            bev = jnp.zeros((_L,), jnp.int32)
            for e in range(_E):
                ce = jnp.max(v_cnts[e])
                nb = (ce + (_TB - 1)) // _TB
                acc = acc + nb
                bev = bev + jnp.where(bid >= acc, 1, 0)
            v_be[pl.ds(j * _L, _L)] = jnp.minimum(bev, _E - 1)
        pltpu.sync_copy(v_be, be_hbm)

    plsc.subcore_barrier()

    @pl.when(core0 & ((sid == 8) | (sid == 9)))
    def _merge():
        which = sid - 8
        pltpu.sync_copy(sh_q.at[which], v_qacc)

        def mbody(k, _):
            s = v_qacc[0, pl.ds(k * _L, _L)]
            for e in range(1, _E):
                s = s + v_qacc[e, pl.ds(k * _L, _L)]
            v_q1[pl.ds(k * _L, _L)] = s
            return 0

        lax.fori_loop(0, n_vregs, mbody, 0)

        @pl.when(which == 0)
        def _w1():
            pltpu.sync_copy(v_q1, q1_hbm)

        @pl.when(which == 1)
        def _w2():
            pltpu.sync_copy(v_q1, q2_hbm)


def _expert_body(be_ref, xs_ref, w1_ref, b1_ref, w2_ref, b2_ref, w3_ref,
                 b3_ref, sw_ref, ys_ref):
    del be_ref
    dn = (((1,), (1,)), ((), ()))
    bf = jnp.bfloat16
    xb = xs_ref[...].astype(bf)
    h1 = jax.lax.dot_general(xb, w1_ref[0].astype(bf), dn,
                             preferred_element_type=jnp.float32)
    h1 = jnp.maximum(h1 + b1_ref[0], 0.0).astype(bf)
    h2 = jax.lax.dot_general(h1, w2_ref[0].astype(bf), dn,
                             preferred_element_type=jnp.float32)
    h2 = jnp.maximum(h2 + b2_ref[0], 0.0).astype(bf)
    ob = jax.lax.dot_general(h2, w3_ref[0].astype(bf), dn,
                             preferred_element_type=jnp.float32)
    ys_ref[...] = (ob + b3_ref[0]) * sw_ref[...]


def kernel(x, Wg, W1, b1, W2, b2, W3, b3):
    Bv, Sv, D = x.shape
    N = Bv * Sv
    E, H = W1.shape[0], W1.shape[1]
    O = W3.shape[1]
    xf = x.reshape(N, D)
    GMAX = (N * _K) // _TB + E
    GPAD = ((GMAX + _L - 1) // _L) * _L
    NS = GMAX * _TB

    wgp = jnp.zeros((128, D), jnp.float32).at[:E].set(Wg)
    gates128, lb, i1c, i2c, g1c, g2c = pl.pallas_call(
        functools.partial(_router_body, n_tokens=N),
        grid=(N // _RB,),
        in_specs=[
            pl.BlockSpec((_RB, D), lambda i: (i, 0)),
            pl.BlockSpec((128, D), lambda i: (0, 0)),
        ],
        out_specs=[
            pl.BlockSpec((_RB, 128), lambda i: (i, 0)),
            pl.BlockSpec((1, 1), lambda i: (0, 0)),
            pl.BlockSpec((_RB, 1), lambda i: (i, 0)),
            pl.BlockSpec((_RB, 1), lambda i: (i, 0)),
            pl.BlockSpec((_RB, 1), lambda i: (i, 0)),
            pl.BlockSpec((_RB, 1), lambda i: (i, 0)),
        ],
        out_shape=[
            jax.ShapeDtypeStruct((N, 128), jnp.float32),
            jax.ShapeDtypeStruct((1, 1), jnp.float32),
            jax.ShapeDtypeStruct((N, 1), jnp.int32),
            jax.ShapeDtypeStruct((N, 1), jnp.int32),
            jax.ShapeDtypeStruct((N, 1), jnp.float32),
            jax.ShapeDtypeStruct((N, 1), jnp.float32),
        ],
        scratch_shapes=[pltpu.VMEM((2, 128), jnp.float32)],
    )(xf, wgp)
    gates = gates128[:, :E]
    lb_loss = lb.reshape(())

    cbuf = N + _TB
    disp = pl.kernel(
        functools.partial(_dispatch_body, n_tokens=N, gmax_pad=GPAD),
        out_type=[
            jax.ShapeDtypeStruct((NS,), jnp.int32),
            jax.ShapeDtypeStruct((NS,), jnp.float32),
            jax.ShapeDtypeStruct((N,), jnp.int32),
            jax.ShapeDtypeStruct((N,), jnp.int32),
            jax.ShapeDtypeStruct((GPAD,), jnp.int32),
        ],
        mesh=plsc.VectorSubcoreMesh(core_axis_name="c", subcore_axis_name="s"),
        compiler_params=pltpu.CompilerParams(needs_layout_passes=False),
        scratch_types=[
            pltpu.VMEM((N,), jnp.int32),
            pltpu.VMEM((N,), jnp.int32),
            pltpu.VMEM((N,), jnp.float32),
            pltpu.VMEM((N,), jnp.float32),
            pltpu.VMEM((cbuf,), jnp.int32),
            pltpu.VMEM((cbuf,), jnp.float32),
            pltpu.VMEM((N,), jnp.int32),
            pltpu.VMEM((N,), jnp.int32),
            pltpu.VMEM((_L,), jnp.int32),
            pltpu.VMEM((_L, _L), jnp.int32),
            pltpu.VMEM((_E, N), jnp.int32),
            pltpu.VMEM((GPAD,), jnp.int32),
            pltpu.VMEM_SHARED((_L, _L), jnp.int32),
            pltpu.VMEM_SHARED((2, _E, N), jnp.int32),
        ],
    )
    slot_token, slot_weight, q1, q2, be = disp(
        i1c.reshape(N), i2c.reshape(N), g1c.reshape(N), g2c.reshape(N))

    xs = jnp.take(xf, slot_token, axis=0, mode="clip")
    sw2d = slot_weight.reshape(NS, 1)
    grid_spec = pltpu.PrefetchScalarGridSpec(
        num_scalar_prefetch=1,
        grid=(GMAX,),
        in_specs=[
            pl.BlockSpec((_TB, D), lambda g, be_r: (g, 0)),
            pl.BlockSpec((1, H, D), lambda g, be_r: (be_r[g], 0, 0)),
            pl.BlockSpec((1, 1, H), lambda g, be_r: (be_r[g], 0, 0)),
            pl.BlockSpec((1, H, H), lambda g, be_r: (be_r[g], 0, 0)),
            pl.BlockSpec((1, 1, H), lambda g, be_r: (be_r[g], 0, 0)),
            pl.BlockSpec((1, O, H), lambda g, be_r: (be_r[g], 0, 0)),
            pl.BlockSpec((1, 1, O), lambda g, be_r: (be_r[g], 0, 0)),
            pl.BlockSpec((_TB, 1), lambda g, be_r: (g, 0)),
        ],
        out_specs=pl.BlockSpec((_TB, O), lambda g, be_r: (g, 0)),
    )
    ys = pl.pallas_call(
        _expert_body,
        grid_spec=grid_spec,
        out_shape=jax.ShapeDtypeStruct((NS, O), jnp.float32),
        compiler_params=pltpu.CompilerParams(
            dimension_semantics=("arbitrary",)),
    )(be[:GMAX], xs, W1, b1.reshape(E, 1, H), W2, b2.reshape(E, 1, H), W3,
      b3.reshape(E, 1, O), sw2d)

    out = (jnp.take(ys, q1, axis=0, mode="clip")
           + jnp.take(ys, q2, axis=0, mode="clip"))
    return out.reshape(Bv, Sv, O), gates, lb_loss


# speedup vs baseline: 1.8022x; 1.2840x over previous
import functools

import jax
import jax.numpy as jnp
from jax import lax
from jax.experimental import pallas as pl
from jax.experimental.pallas import tpu as pltpu
from jax.experimental.pallas import tpu_sc as plsc

_E = 8
_K = 2
_TB = 256
_RB = 256
_L = 16


def _router_body(xb_ref, wgp_ref, gates_ref, lb_ref, i1_ref, i2_ref,
                 g1_ref, g2_ref, acc_ref, *, n_tokens):
    i = pl.program_id(0)
    nsteps = n_tokens // _RB
    lg = jax.lax.dot_general(
        xb_ref[...], wgp_ref[...], (((1,), (1,)), ((), ())),
        preferred_element_type=jnp.float32)
    lane = jax.lax.broadcasted_iota(jnp.int32, lg.shape, 1)
    neg = jnp.float32(-1e30)
    lgm = jnp.where(lane < _E, lg, neg)
    m1 = jnp.max(lgm, axis=1, keepdims=True)
    i1 = jnp.min(jnp.where(lgm == m1, lane, 127), axis=1, keepdims=True)
    lg2 = jnp.where(lane == i1, neg, lgm)
    m2 = jnp.max(lg2, axis=1, keepdims=True)
    i2 = jnp.min(jnp.where(lg2 == m2, lane, 127), axis=1, keepdims=True)
    t = jnp.exp(m2 - m1)
    g1 = 1.0 / (1.0 + t)
    g2 = t / (1.0 + t)
    gates = jnp.where(lane == i1, g1, 0.0) + jnp.where(lane == i2, g2, 0.0)
    gates_ref[...] = gates
    i1_ref[...] = i1
    i2_ref[...] = i2
    g1_ref[...] = g1
    g2_ref[...] = g2

    @pl.when(i == 0)
    def _init():
        acc_ref[...] = jnp.zeros_like(acc_ref)

    acc_ref[0:1, :] += jnp.sum(gates, axis=0, keepdims=True)
    acc_ref[1:2, :] += jnp.sum((gates > 0).astype(jnp.float32), axis=0,
                               keepdims=True)

    @pl.when(i == nsteps - 1)
    def _fin():
        scale = jnp.float32(_E / (float(n_tokens) * float(n_tokens)))
        lb = jnp.sum(acc_ref[0:1, :] * acc_ref[1:2, :], axis=1, keepdims=True)
        lb_ref[...] = lb * scale


def _dispatch_body(i1_hbm, i2_hbm, g1_hbm, g2_hbm,
                   st_hbm, sw_hbm, q1_hbm, q2_hbm, be_hbm, tot_hbm,
                   v_i1, v_i2, v_g1, v_g2, v_st, v_sw, v_q1, v_q2,
                   v_cnt, v_cnts, v_qacc, v_be, sh_cnt, sh_q,
                   *, n_tokens, gmax_pad):
    n_vregs = n_tokens // _L
    cid = lax.axis_index("c")
    sid = lax.axis_index("s")
    core0 = cid == 0
    is_expert = core0 & (sid < _E)
    iota = lax.iota(jnp.int32, _L)

    @pl.when(is_expert)
    def _count():
        pltpu.sync_copy(i1_hbm, v_i1)
        pltpu.sync_copy(i2_hbm, v_i2)
        pltpu.sync_copy(g1_hbm, v_g1)
        pltpu.sync_copy(g2_hbm, v_g2)

        def body(k, cnt):
            a = v_i1[pl.ds(k * _L, _L)]
            b = v_i2[pl.ds(k * _L, _L)]
            c1 = jnp.where(a == sid, 1, 0)
            c2 = jnp.where(b == sid, 1, 0)
            return cnt + jnp.sum(c1 + c2)

        cnt = lax.fori_loop(0, n_vregs, body, jnp.int32(0))
        v_cnt[...] = jnp.broadcast_to(cnt, (_L,))
        pltpu.sync_copy(v_cnt, sh_cnt.at[sid])

    plsc.subcore_barrier()

    @pl.when(core0 & (sid < 11))
    def _layout():
        pltpu.sync_copy(sh_cnt, v_cnts)

    plsc.subcore_barrier()

    @pl.when(is_expert)
    def _place():
        base = jnp.int32(0)
        for e in range(_E):
            ce = jnp.max(v_cnts[e])
            nb = (ce + (_TB - 1)) // _TB
            base = base + jnp.where(e < sid, nb * _TB, 0)

        def scan(k, off, src_ref, wsrc_ref, q_ref):
            a = src_ref[pl.ds(k * _L, _L)]
            w = wsrc_ref[pl.ds(k * _L, _L)]
            m = a == sid
            pref = jnp.cumsum(jnp.where(m, 1, 0))
            loc = off + pref - 1
            tokv = iota + k * _L
            plsc.store_scatter(v_st, [loc], tokv, mask=m)
            plsc.store_scatter(v_sw, [loc], w, mask=m)
            q_ref[pl.ds(k * _L, _L)] = jnp.where(m, base + loc, 0)
            return off + jnp.max(pref)

        off = lax.fori_loop(
            0, n_vregs, lambda k, o: scan(k, o, v_i1, v_g1, v_q1),
            jnp.int32(0))
        off = lax.fori_loop(
            0, n_vregs, lambda k, o: scan(k, o, v_i2, v_g2, v_q2), off)

        nblk = (off + (_TB - 1)) // _TB
        for j in range(n_tokens // _TB):
            @pl.when(j < nblk)
            def _wr():
                dst = pl.multiple_of(base + j * _TB, _TB)
                pltpu.sync_copy(v_st.at[pl.ds(j * _TB, _TB)],
                                st_hbm.at[pl.ds(dst, _TB)])
                pltpu.sync_copy(v_sw.at[pl.ds(j * _TB, _TB)],
                                sw_hbm.at[pl.ds(dst, _TB)])

        pltpu.sync_copy(v_q1, sh_q.at[0, sid])
        pltpu.sync_copy(v_q2, sh_q.at[1, sid])

    @pl.when(core0 & (sid == 10))
    def _blockmap():
        for j in range(gmax_pad // _L):
            bid = iota + j * _L
            acc = jnp.int32(0)
            bev = jnp.zeros((_L,), jnp.int32)
            for e in range(_E):
                ce = jnp.max(v_cnts[e])
                nb = (ce + (_TB - 1)) // _TB
                acc = acc + nb
                bev = bev + jnp.where(bid >= acc, 1, 0)
            v_be[pl.ds(j * _L, _L)] = jnp.minimum(bev, _E - 1)
        pltpu.sync_copy(v_be, be_hbm)
        tot = jnp.int32(0)
        for e in range(_E):
            ce = jnp.max(v_cnts[e])
            tot = tot + (ce + (_TB - 1)) // _TB
        v_cnt[...] = jnp.broadcast_to(tot, (_L,))
        pltpu.sync_copy(v_cnt, tot_hbm)

    plsc.subcore_barrier()

    @pl.when(core0 & ((sid == 8) | (sid == 9)))
    def _merge():
        which = sid - 8
        pltpu.sync_copy(sh_q.at[which], v_qacc)

        def mbody(k, _):
            s = v_qacc[0, pl.ds(k * _L, _L)]
            for e in range(1, _E):
                s = s + v_qacc[e, pl.ds(k * _L, _L)]
            v_q1[pl.ds(k * _L, _L)] = s
            return 0

        lax.fori_loop(0, n_vregs, mbody, 0)

        @pl.when(which == 0)
        def _w1():
            pltpu.sync_copy(v_q1, q1_hbm)

        @pl.when(which == 1)
        def _w2():
            pltpu.sync_copy(v_q1, q2_hbm)


def _expert_body(be_ref, tot_ref, xs_ref, w1_ref, b1_ref, w2_ref, b2_ref,
                 w3_ref, b3_ref, sw_ref, ys_ref):
    del be_ref

    @pl.when(pl.program_id(0) < tot_ref[0])
    def _active():
        dn = (((1,), (1,)), ((), ()))
        bf = jnp.bfloat16
        xb = xs_ref[...].astype(bf)
        h1 = jax.lax.dot_general(xb, w1_ref[0].astype(bf), dn,
                                 preferred_element_type=jnp.float32)
        h1 = jnp.maximum(h1 + b1_ref[0], 0.0).astype(bf)
        h2 = jax.lax.dot_general(h1, w2_ref[0].astype(bf), dn,
                                 preferred_element_type=jnp.float32)
        h2 = jnp.maximum(h2 + b2_ref[0], 0.0).astype(bf)
        ob = jax.lax.dot_general(h2, w3_ref[0].astype(bf), dn,
                                 preferred_element_type=jnp.float32)
        ys_ref[...] = (ob + b3_ref[0]) * sw_ref[...]


def kernel(x, Wg, W1, b1, W2, b2, W3, b3):
    Bv, Sv, D = x.shape
    N = Bv * Sv
    E, H = W1.shape[0], W1.shape[1]
    O = W3.shape[1]
    xf = x.reshape(N, D)
    GMAX = (N * _K) // _TB + E
    GPAD = ((GMAX + _L - 1) // _L) * _L
    NS = GMAX * _TB

    wgp = jnp.zeros((128, D), jnp.float32).at[:E].set(Wg)
    gates128, lb, i1c, i2c, g1c, g2c = pl.pallas_call(
        functools.partial(_router_body, n_tokens=N),
        grid=(N // _RB,),
        in_specs=[
            pl.BlockSpec((_RB, D), lambda i: (i, 0)),
            pl.BlockSpec((128, D), lambda i: (0, 0)),
        ],
        out_specs=[
            pl.BlockSpec((_RB, 128), lambda i: (i, 0)),
            pl.BlockSpec((1, 1), lambda i: (0, 0)),
            pl.BlockSpec((_RB, 1), lambda i: (i, 0)),
            pl.BlockSpec((_RB, 1), lambda i: (i, 0)),
            pl.BlockSpec((_RB, 1), lambda i: (i, 0)),
            pl.BlockSpec((_RB, 1), lambda i: (i, 0)),
        ],
        out_shape=[
            jax.ShapeDtypeStruct((N, 128), jnp.float32),
            jax.ShapeDtypeStruct((1, 1), jnp.float32),
            jax.ShapeDtypeStruct((N, 1), jnp.int32),
            jax.ShapeDtypeStruct((N, 1), jnp.int32),
            jax.ShapeDtypeStruct((N, 1), jnp.float32),
            jax.ShapeDtypeStruct((N, 1), jnp.float32),
        ],
        scratch_shapes=[pltpu.VMEM((2, 128), jnp.float32)],
    )(xf, wgp)
    gates = gates128[:, :E]
    lb_loss = lb.reshape(())

    cbuf = N + _TB
    disp = pl.kernel(
        functools.partial(_dispatch_body, n_tokens=N, gmax_pad=GPAD),
        out_type=[
            jax.ShapeDtypeStruct((NS,), jnp.int32),
            jax.ShapeDtypeStruct((NS,), jnp.float32),
            jax.ShapeDtypeStruct((N,), jnp.int32),
            jax.ShapeDtypeStruct((N,), jnp.int32),
            jax.ShapeDtypeStruct((GPAD,), jnp.int32),
            jax.ShapeDtypeStruct((_L,), jnp.int32),
        ],
        mesh=plsc.VectorSubcoreMesh(core_axis_name="c", subcore_axis_name="s"),
        compiler_params=pltpu.CompilerParams(needs_layout_passes=False),
        scratch_types=[
            pltpu.VMEM((N,), jnp.int32),
            pltpu.VMEM((N,), jnp.int32),
            pltpu.VMEM((N,), jnp.float32),
            pltpu.VMEM((N,), jnp.float32),
            pltpu.VMEM((cbuf,), jnp.int32),
            pltpu.VMEM((cbuf,), jnp.float32),
            pltpu.VMEM((N,), jnp.int32),
            pltpu.VMEM((N,), jnp.int32),
            pltpu.VMEM((_L,), jnp.int32),
            pltpu.VMEM((_L, _L), jnp.int32),
            pltpu.VMEM((_E, N), jnp.int32),
            pltpu.VMEM((GPAD,), jnp.int32),
            pltpu.VMEM_SHARED((_L, _L), jnp.int32),
            pltpu.VMEM_SHARED((2, _E, N), jnp.int32),
        ],
    )
    slot_token, slot_weight, q1, q2, be, tot = disp(
        i1c.reshape(N), i2c.reshape(N), g1c.reshape(N), g2c.reshape(N))

    xs = jnp.take(xf, slot_token, axis=0, mode="clip")
    sw2d = slot_weight.reshape(NS, 1)
    grid_spec = pltpu.PrefetchScalarGridSpec(
        num_scalar_prefetch=2,
        grid=(GMAX,),
        in_specs=[
            pl.BlockSpec((_TB, D), lambda g, be_r, t_r: (g, 0)),
            pl.BlockSpec((1, H, D), lambda g, be_r, t_r: (be_r[g], 0, 0)),
            pl.BlockSpec((1, 1, H), lambda g, be_r, t_r: (be_r[g], 0, 0)),
            pl.BlockSpec((1, H, H), lambda g, be_r, t_r: (be_r[g], 0, 0)),
            pl.BlockSpec((1, 1, H), lambda g, be_r, t_r: (be_r[g], 0, 0)),
            pl.BlockSpec((1, O, H), lambda g, be_r, t_r: (be_r[g], 0, 0)),
            pl.BlockSpec((1, 1, O), lambda g, be_r, t_r: (be_r[g], 0, 0)),
            pl.BlockSpec((_TB, 1), lambda g, be_r, t_r: (g, 0)),
        ],
        out_specs=pl.BlockSpec((_TB, O), lambda g, be_r, t_r: (g, 0)),
    )
    ys = pl.pallas_call(
        _expert_body,
        grid_spec=grid_spec,
        out_shape=jax.ShapeDtypeStruct((NS, O), jnp.float32),
        compiler_params=pltpu.CompilerParams(
            dimension_semantics=("arbitrary",)),
    )(be[:GMAX], tot, xs, W1, b1.reshape(E, 1, H), W2, b2.reshape(E, 1, H),
      W3, b3.reshape(E, 1, O), sw2d)

    out = (jnp.take(ys, q1, axis=0, mode="clip")
           + jnp.take(ys, q2, axis=0, mode="clip"))
    return out.reshape(Bv, Sv, O), gates, lb_loss


# speedup vs baseline: 1.8189x; 1.0093x over previous
import functools

import jax
import jax.numpy as jnp
from jax import lax
from jax.experimental import pallas as pl
from jax.experimental.pallas import tpu as pltpu
from jax.experimental.pallas import tpu_sc as plsc

_E = 8
_K = 2
_TB = 256
_RB = 256
_L = 16


def _router_body(xb_ref, wgp_ref, gates_ref, lb_ref, i1_ref, i2_ref,
                 g1_ref, g2_ref, acc_ref, *, n_tokens):
    i = pl.program_id(0)
    nsteps = n_tokens // _RB
    lg = jax.lax.dot_general(
        xb_ref[...], wgp_ref[...], (((1,), (1,)), ((), ())),
        preferred_element_type=jnp.float32)
    lane = jax.lax.broadcasted_iota(jnp.int32, lg.shape, 1)
    neg = jnp.float32(-1e30)
    lgm = jnp.where(lane < _E, lg, neg)
    m1 = jnp.max(lgm, axis=1, keepdims=True)
    i1 = jnp.min(jnp.where(lgm == m1, lane, 127), axis=1, keepdims=True)
    lg2 = jnp.where(lane == i1, neg, lgm)
    m2 = jnp.max(lg2, axis=1, keepdims=True)
    i2 = jnp.min(jnp.where(lg2 == m2, lane, 127), axis=1, keepdims=True)
    t = jnp.exp(m2 - m1)
    g1 = 1.0 / (1.0 + t)
    g2 = t / (1.0 + t)
    gates = jnp.where(lane == i1, g1, 0.0) + jnp.where(lane == i2, g2, 0.0)
    gates_ref[...] = gates
    i1_ref[...] = i1
    i2_ref[...] = i2
    g1_ref[...] = g1
    g2_ref[...] = g2

    @pl.when(i == 0)
    def _init():
        acc_ref[...] = jnp.zeros_like(acc_ref)

    acc_ref[0:1, :] += jnp.sum(gates, axis=0, keepdims=True)
    acc_ref[1:2, :] += jnp.sum((gates > 0).astype(jnp.float32), axis=0,
                               keepdims=True)

    @pl.when(i == nsteps - 1)
    def _fin():
        scale = jnp.float32(_E / (float(n_tokens) * float(n_tokens)))
        lb = jnp.sum(acc_ref[0:1, :] * acc_ref[1:2, :], axis=1, keepdims=True)
        lb_ref[...] = lb * scale


def _dispatch_body(i1_hbm, i2_hbm, g1_hbm, g2_hbm,
                   st_hbm, sw_hbm, q1_hbm, q2_hbm, be_hbm, tot_hbm,
                   v_i1, v_i2, v_g1, v_g2, v_st, v_sw, v_q1, v_q2,
                   v_cnt, v_cnts, v_qacc, v_be, sh_cnt, sh_q,
                   *, n_tokens, gmax_pad):
    n_vregs = n_tokens // _L
    cid = lax.axis_index("c")
    sid = lax.axis_index("s")
    core0 = cid == 0
    is_expert = core0 & (sid < _E)
    iota = lax.iota(jnp.int32, _L)

    @pl.when(is_expert)
    def _count():
        pltpu.sync_copy(i1_hbm, v_i1)
        pltpu.sync_copy(i2_hbm, v_i2)
        pltpu.sync_copy(g1_hbm, v_g1)
        pltpu.sync_copy(g2_hbm, v_g2)

        def body(k, cnt):
            a = v_i1[pl.ds(k * _L, _L)]
            b = v_i2[pl.ds(k * _L, _L)]
            c1 = jnp.where(a == sid, 1, 0)
            c2 = jnp.where(b == sid, 1, 0)
            return cnt + jnp.sum(c1 + c2)

        cnt = lax.fori_loop(0, n_vregs, body, jnp.int32(0))
        v_cnt[...] = jnp.broadcast_to(cnt, (_L,))
        pltpu.sync_copy(v_cnt, sh_cnt.at[sid])

    plsc.subcore_barrier()

    @pl.when(core0 & (sid < 11))
    def _layout():
        pltpu.sync_copy(sh_cnt, v_cnts)

    plsc.subcore_barrier()

    @pl.when(is_expert)
    def _place():
        base = jnp.int32(0)
        for e in range(_E):
            ce = jnp.max(v_cnts[e])
            nb = (ce + (_TB - 1)) // _TB
            base = base + jnp.where(e < sid, nb * _TB, 0)

        def scan(k, off, src_ref, wsrc_ref, q_ref):
            a = src_ref[pl.ds(k * _L, _L)]
            w = wsrc_ref[pl.ds(k * _L, _L)]
            m = a == sid
            pref = jnp.cumsum(jnp.where(m, 1, 0))
            loc = off + pref - 1
            tokv = iota + k * _L
            plsc.store_scatter(v_st, [loc], tokv, mask=m)
            plsc.store_scatter(v_sw, [loc], w, mask=m)
            q_ref[pl.ds(k * _L, _L)] = jnp.where(m, base + loc, 0)
            return off + jnp.max(pref)

        off = lax.fori_loop(
            0, n_vregs, lambda k, o: scan(k, o, v_i1, v_g1, v_q1),
            jnp.int32(0))
        off = lax.fori_loop(
            0, n_vregs, lambda k, o: scan(k, o, v_i2, v_g2, v_q2), off)

        nblk = (off + (_TB - 1)) // _TB
        for j in range(n_tokens // _TB):
            @pl.when(j < nblk)
            def _wr():
                dst = pl.multiple_of(base + j * _TB, _TB)
                pltpu.sync_copy(v_st.at[pl.ds(j * _TB, _TB)],
                                st_hbm.at[pl.ds(dst, _TB)])
                pltpu.sync_copy(v_sw.at[pl.ds(j * _TB, _TB)],
                                sw_hbm.at[pl.ds(dst, _TB)])

        pltpu.sync_copy(v_q1, sh_q.at[0, sid])
        pltpu.sync_copy(v_q2, sh_q.at[1, sid])

    @pl.when(core0 & (sid == 10))
    def _blockmap():
        for j in range(gmax_pad // _L):
            bid = iota + j * _L
            acc = jnp.int32(0)
            bev = jnp.zeros((_L,), jnp.int32)
            for e in range(_E):
                ce = jnp.max(v_cnts[e])
                nb = (ce + (_TB - 1)) // _TB
                acc = acc + nb
                bev = bev + jnp.where(bid >= acc, 1, 0)
            v_be[pl.ds(j * _L, _L)] = jnp.minimum(bev, _E - 1)
        pltpu.sync_copy(v_be, be_hbm)
        tot = jnp.int32(0)
        for e in range(_E):
            ce = jnp.max(v_cnts[e])
            tot = tot + (ce + (_TB - 1)) // _TB
        v_cnt[...] = jnp.broadcast_to(tot, (_L,))
        pltpu.sync_copy(v_cnt, tot_hbm)

    plsc.subcore_barrier()

    @pl.when(core0 & ((sid == 8) | (sid == 9)))
    def _merge():
        which = sid - 8
        pltpu.sync_copy(sh_q.at[which], v_qacc)

        def mbody(k, _):
            s = v_qacc[0, pl.ds(k * _L, _L)]
            for e in range(1, _E):
                s = s + v_qacc[e, pl.ds(k * _L, _L)]
            v_q1[pl.ds(k * _L, _L)] = s
            return 0

        lax.fori_loop(0, n_vregs, mbody, 0)

        @pl.when(which == 0)
        def _w1():
            pltpu.sync_copy(v_q1, q1_hbm)

        @pl.when(which == 1)
        def _w2():
            pltpu.sync_copy(v_q1, q2_hbm)


def _expert_body(be_ref, tot_ref, xs_ref, w1_ref, b1_ref, w2_ref, b2_ref,
                 w3_ref, b3_ref, sw_ref, ys_ref):
    del be_ref

    @pl.when(pl.program_id(0) < tot_ref[0])
    def _active():
        dn = (((1,), (1,)), ((), ()))
        bf = jnp.bfloat16
        xb = xs_ref[...]
        h1 = jax.lax.dot_general(xb, w1_ref[0].astype(bf), dn,
                                 preferred_element_type=jnp.float32)
        h1 = jnp.maximum(h1 + b1_ref[0], 0.0).astype(bf)
        h2 = jax.lax.dot_general(h1, w2_ref[0].astype(bf), dn,
                                 preferred_element_type=jnp.float32)
        h2 = jnp.maximum(h2 + b2_ref[0], 0.0).astype(bf)
        ob = jax.lax.dot_general(h2, w3_ref[0].astype(bf), dn,
                                 preferred_element_type=jnp.float32)
        ys_ref[...] = (ob + b3_ref[0]) * sw_ref[...]


def kernel(x, Wg, W1, b1, W2, b2, W3, b3):
    Bv, Sv, D = x.shape
    N = Bv * Sv
    E, H = W1.shape[0], W1.shape[1]
    O = W3.shape[1]
    xf = x.reshape(N, D)
    GMAX = (N * _K) // _TB + E
    GPAD = ((GMAX + _L - 1) // _L) * _L
    NS = GMAX * _TB

    wgp = jnp.zeros((128, D), jnp.float32).at[:E].set(Wg)
    gates128, lb, i1c, i2c, g1c, g2c = pl.pallas_call(
        functools.partial(_router_body, n_tokens=N),
        grid=(N // _RB,),
        in_specs=[
            pl.BlockSpec((_RB, D), lambda i: (i, 0)),
            pl.BlockSpec((128, D), lambda i: (0, 0)),
        ],
        out_specs=[
            pl.BlockSpec((_RB, 128), lambda i: (i, 0)),
            pl.BlockSpec((1, 1), lambda i: (0, 0)),
            pl.BlockSpec((_RB, 1), lambda i: (i, 0)),
            pl.BlockSpec((_RB, 1), lambda i: (i, 0)),
            pl.BlockSpec((_RB, 1), lambda i: (i, 0)),
            pl.BlockSpec((_RB, 1), lambda i: (i, 0)),
        ],
        out_shape=[
            jax.ShapeDtypeStruct((N, 128), jnp.float32),
            jax.ShapeDtypeStruct((1, 1), jnp.float32),
            jax.ShapeDtypeStruct((N, 1), jnp.int32),
            jax.ShapeDtypeStruct((N, 1), jnp.int32),
            jax.ShapeDtypeStruct((N, 1), jnp.float32),
            jax.ShapeDtypeStruct((N, 1), jnp.float32),
        ],
        scratch_shapes=[pltpu.VMEM((2, 128), jnp.float32)],
    )(xf, wgp)
    gates = gates128[:, :E]
    lb_loss = lb.reshape(())

    cbuf = N + _TB
    disp = pl.kernel(
        functools.partial(_dispatch_body, n_tokens=N, gmax_pad=GPAD),
        out_type=[
            jax.ShapeDtypeStruct((NS,), jnp.int32),
            jax.ShapeDtypeStruct((NS,), jnp.float32),
            jax.ShapeDtypeStruct((N,), jnp.int32),
            jax.ShapeDtypeStruct((N,), jnp.int32),
            jax.ShapeDtypeStruct((GPAD,), jnp.int32),
            jax.ShapeDtypeStruct((_L,), jnp.int32),
        ],
        mesh=plsc.VectorSubcoreMesh(core_axis_name="c", subcore_axis_name="s"),
        compiler_params=pltpu.CompilerParams(needs_layout_passes=False),
        scratch_types=[
            pltpu.VMEM((N,), jnp.int32),
            pltpu.VMEM((N,), jnp.int32),
            pltpu.VMEM((N,), jnp.float32),
            pltpu.VMEM((N,), jnp.float32),
            pltpu.VMEM((cbuf,), jnp.int32),
            pltpu.VMEM((cbuf,), jnp.float32),
            pltpu.VMEM((N,), jnp.int32),
            pltpu.VMEM((N,), jnp.int32),
            pltpu.VMEM((_L,), jnp.int32),
            pltpu.VMEM((_L, _L), jnp.int32),
            pltpu.VMEM((_E, N), jnp.int32),
            pltpu.VMEM((GPAD,), jnp.int32),
            pltpu.VMEM_SHARED((_L, _L), jnp.int32),
            pltpu.VMEM_SHARED((2, _E, N), jnp.int32),
        ],
    )
    slot_token, slot_weight, q1, q2, be, tot = disp(
        i1c.reshape(N), i2c.reshape(N), g1c.reshape(N), g2c.reshape(N))

    xs = jnp.take(xf.astype(jnp.bfloat16), slot_token, axis=0,
                  mode="clip")
    sw2d = slot_weight.reshape(NS, 1)
    grid_spec = pltpu.PrefetchScalarGridSpec(
        num_scalar_prefetch=2,
        grid=(GMAX,),
        in_specs=[
            pl.BlockSpec((_TB, D), lambda g, be_r, t_r: (g, 0)),
            pl.BlockSpec((1, H, D), lambda g, be_r, t_r: (be_r[g], 0, 0)),
            pl.BlockSpec((1, 1, H), lambda g, be_r, t_r: (be_r[g], 0, 0)),
            pl.BlockSpec((1, H, H), lambda g, be_r, t_r: (be_r[g], 0, 0)),
            pl.BlockSpec((1, 1, H), lambda g, be_r, t_r: (be_r[g], 0, 0)),
            pl.BlockSpec((1, O, H), lambda g, be_r, t_r: (be_r[g], 0, 0)),
            pl.BlockSpec((1, 1, O), lambda g, be_r, t_r: (be_r[g], 0, 0)),
            pl.BlockSpec((_TB, 1), lambda g, be_r, t_r: (g, 0)),
        ],
        out_specs=pl.BlockSpec((_TB, O), lambda g, be_r, t_r: (g, 0)),
    )
    ys = pl.pallas_call(
        _expert_body,
        grid_spec=grid_spec,
        out_shape=jax.ShapeDtypeStruct((NS, O), jnp.float32),
        compiler_params=pltpu.CompilerParams(
            dimension_semantics=("arbitrary",)),
    )(be[:GMAX], tot, xs, W1, b1.reshape(E, 1, H), W2, b2.reshape(E, 1, H),
      W3, b3.reshape(E, 1, O), sw2d)

    out = (jnp.take(ys, q1, axis=0, mode="clip")
           + jnp.take(ys, q2, axis=0, mode="clip"))
    return out.reshape(Bv, Sv, O), gates, lb_loss
